# Initial kernel scaffold; baseline (speedup 1.0000x reference)
import jax
import jax.numpy as jnp
from jax.experimental import pallas as pl


def kernel(x, W_color, b_color, Wb1, bb1, bw1, Wb2, bb2, bw2, Wb3, bb3, bw3, We, be, gamma, beta):
    raise NotImplementedError("write your pallas kernel here")


# speedup vs baseline: 2.2592x; 2.2592x over previous
import jax
import jax.numpy as jnp
from jax.experimental import pallas as pl
from jax.experimental.pallas import tpu as pltpu

_S, _D, _SD = 2048, 1024, 64
_ND, _NS, _NC = 4, 4, 4
_NB, _BSZ = 4, 16
_K = _ND * _NS * _NC


def _group_mats(width, group):
    i = jax.lax.broadcasted_iota(jnp.int32, (width, width), 0)
    j = jax.lax.broadcasted_iota(jnp.int32, (width, width), 1)
    g = jnp.where((i // group) == (j // group), 1.0, 0.0).astype(jnp.bfloat16)
    gi = jax.lax.broadcasted_iota(jnp.int32, (width // group, width), 0)
    gj = jax.lax.broadcasted_iota(jnp.int32, (width // group, width), 1)
    b = jnp.where(gi == (gj // group), 1.0, 0.0).astype(jnp.bfloat16)
    return g, b


def _body(x_ref, wcol_ref, bcol_ref, wb1_ref, bb1_ref, bw1_ref,
          wb2_ref, bb2_ref, bw2_ref, wb3_ref, bb3_ref, bw3_ref,
          we_ref, be_ref, gamma_ref, beta_ref,
          out_ref, xb_scr, j3_scr):
    k = pl.program_id(0)

    @pl.when(k == 0)
    def _routing():
        xb = x_ref[...].astype(jnp.bfloat16)
        xb_scr[...] = xb
        wcol = wcol_ref[...].astype(jnp.bfloat16)
        color = jnp.tanh(
            jnp.dot(xb, wcol, preferred_element_type=jnp.float32)
            + bcol_ref[...])
        color_b = color.astype(jnp.bfloat16)

        def chroma(wb_ref, bb_ref, bw_ref, kl):
            bw = bw_ref[...]
            e = jnp.exp(bw - jnp.max(bw, axis=-1, keepdims=True))
            w = e / jnp.sum(e, axis=-1, keepdims=True)
            r = jnp.zeros((_S, kl), jnp.float32)
            for n in range(_NB):
                band = color_b[:, n * _BSZ:(n + 1) * _BSZ]
                wn = wb_ref[n].astype(jnp.bfloat16)
                logit = jnp.dot(band, wn, preferred_element_type=jnp.float32)
                logit = logit + bb_ref[n:n + 1, :]
                r = r + w[:, n:n + 1] * jax.nn.sigmoid(logit)
            return r

        r1 = chroma(wb1_ref, bb1_ref, bw1_ref, _ND)
        r2 = chroma(wb2_ref, bb2_ref, bw2_ref, _ND * _NS)
        r3 = chroma(wb3_ref, bb3_ref, bw3_ref, _K)

        e1 = jnp.exp(r1)
        p1 = e1 / jnp.sum(e1, axis=-1, keepdims=True)

        g16, b16 = _group_mats(_ND * _NS, _NS)
        e2 = jnp.exp(r2)
        s2 = jnp.dot(e2.astype(jnp.bfloat16), g16, preferred_element_type=jnp.float32)
        p2 = e2 / s2
        j2 = jnp.dot(p1.astype(jnp.bfloat16), b16, preferred_element_type=jnp.float32) * p2

        g64, b64 = _group_mats(_K, _NC)
        e3 = jnp.exp(r3)
        s3 = jnp.dot(e3.astype(jnp.bfloat16), g64, preferred_element_type=jnp.float32)
        p3 = e3 / s3
        j3 = jnp.dot(j2.astype(jnp.bfloat16), b64, preferred_element_type=jnp.float32) * p3
        j3_scr[...] = j3.astype(jnp.bfloat16)

    we = we_ref[0].astype(jnp.bfloat16)
    mm = jnp.dot(xb_scr[...], we, preferred_element_type=jnp.float32)
    oh = (jax.lax.broadcasted_iota(jnp.int32, (_K, 1), 0) == k
          ).astype(jnp.bfloat16)
    col = jnp.dot(j3_scr[...], oh, preferred_element_type=jnp.float32)
    contrib = col * mm

    @pl.when(k == 0)
    def _first():
        out_ref[...] = contrib

    @pl.when(k > 0)
    def _accum():
        out_ref[...] = out_ref[...] + contrib

    @pl.when(k == _K - 1)
    def _finish():
        h = out_ref[...] + jnp.dot(
            j3_scr[...], be_ref[...].astype(jnp.bfloat16),
            preferred_element_type=jnp.float32)
        y = x_ref[...] + jax.nn.gelu(h)
        mu = jnp.mean(y, axis=-1, keepdims=True)
        var = jnp.mean((y - mu) ** 2, axis=-1, keepdims=True)
        out_ref[...] = ((y - mu) * jax.lax.rsqrt(var + 1e-5)
                        * gamma_ref[...] + beta_ref[...])


def kernel(x, W_color, b_color, Wb1, bb1, bw1, Wb2, bb2, bw2, Wb3, bb3, bw3,
           We, be, gamma, beta):
    x2 = x.reshape(_S, _D)
    out = pl.pallas_call(
        _body,
        grid=(_K,),
        in_specs=[
            pl.BlockSpec((_S, _D), lambda k: (0, 0)),
            pl.BlockSpec((_D, _SD), lambda k: (0, 0)),
            pl.BlockSpec((1, _SD), lambda k: (0, 0)),
            pl.BlockSpec((_NB, _BSZ, _ND), lambda k: (0, 0, 0)),
            pl.BlockSpec((_NB, _ND), lambda k: (0, 0)),
            pl.BlockSpec((1, _NB), lambda k: (0, 0)),
            pl.BlockSpec((_NB, _BSZ, _ND * _NS), lambda k: (0, 0, 0)),
            pl.BlockSpec((_NB, _ND * _NS), lambda k: (0, 0)),
            pl.BlockSpec((1, _NB), lambda k: (0, 0)),
            pl.BlockSpec((_NB, _BSZ, _K), lambda k: (0, 0, 0)),
            pl.BlockSpec((_NB, _K), lambda k: (0, 0)),
            pl.BlockSpec((1, _NB), lambda k: (0, 0)),
            pl.BlockSpec((1, _D, _D), lambda k: (k, 0, 0)),
            pl.BlockSpec((_K, _D), lambda k: (0, 0)),
            pl.BlockSpec((1, _D), lambda k: (0, 0)),
            pl.BlockSpec((1, _D), lambda k: (0, 0)),
        ],
        out_specs=pl.BlockSpec((_S, _D), lambda k: (0, 0)),
        out_shape=jax.ShapeDtypeStruct((_S, _D), jnp.float32),
        scratch_shapes=[
            pltpu.VMEM((_S, _D), jnp.bfloat16),
            pltpu.VMEM((_S, _K), jnp.bfloat16),
        ],
        compiler_params=pltpu.CompilerParams(
            dimension_semantics=("arbitrary",),
        ),
    )(x2, W_color, b_color.reshape(1, _SD), Wb1, bb1, bw1.reshape(1, _NB),
      Wb2, bb2, bw2.reshape(1, _NB), Wb3, bb3, bw3.reshape(1, _NB),
      We, be, gamma.reshape(1, _D), beta.reshape(1, _D))
    return out.reshape(1, _S, _D)


# speedup vs baseline: 3.3394x; 1.4781x over previous
import jax
import jax.numpy as jnp
from jax.experimental import pallas as pl
from jax.experimental.pallas import tpu as pltpu

_S, _D, _SD = 2048, 1024, 64
_ND, _NS, _NC = 4, 4, 4
_NB, _BSZ = 4, 16
_K = _ND * _NS * _NC


def _group_mats(width, group):
    i = jax.lax.broadcasted_iota(jnp.int32, (width, width), 0)
    j = jax.lax.broadcasted_iota(jnp.int32, (width, width), 1)
    g = jnp.where((i // group) == (j // group), 1.0, 0.0).astype(jnp.bfloat16)
    gi = jax.lax.broadcasted_iota(jnp.int32, (width // group, width), 0)
    gj = jax.lax.broadcasted_iota(jnp.int32, (width // group, width), 1)
    b = jnp.where(gi == (gj // group), 1.0, 0.0).astype(jnp.bfloat16)
    return g, b


def _body(x_ref, wcol_ref, bcol_ref, wb1_ref, bb1_ref, bw1_ref,
          wb2_ref, bb2_ref, bw2_ref, wb3_ref, bb3_ref, bw3_ref,
          we_ref, be_ref, gamma_ref, beta_ref,
          out_ref, xb_scr, j3_scr):
    k = pl.program_id(0)

    @pl.when(k == 0)
    def _routing():
        xb = x_ref[...].astype(jnp.bfloat16)
        xb_scr[...] = x_ref[...].astype(jnp.float8_e4m3fn)
        wcol = wcol_ref[...].astype(jnp.bfloat16)
        color = jnp.tanh(
            jnp.dot(xb, wcol, preferred_element_type=jnp.float32)
            + bcol_ref[...])
        color_b = color.astype(jnp.bfloat16)

        def chroma(wb_ref, bb_ref, bw_ref, kl):
            bw = bw_ref[...]
            e = jnp.exp(bw - jnp.max(bw, axis=-1, keepdims=True))
            w = e / jnp.sum(e, axis=-1, keepdims=True)
            r = jnp.zeros((_S, kl), jnp.float32)
            for n in range(_NB):
                band = color_b[:, n * _BSZ:(n + 1) * _BSZ]
                wn = wb_ref[n].astype(jnp.bfloat16)
                logit = jnp.dot(band, wn, preferred_element_type=jnp.float32)
                logit = logit + bb_ref[n:n + 1, :]
                r = r + w[:, n:n + 1] * jax.nn.sigmoid(logit)
            return r

        r1 = chroma(wb1_ref, bb1_ref, bw1_ref, _ND)
        r2 = chroma(wb2_ref, bb2_ref, bw2_ref, _ND * _NS)
        r3 = chroma(wb3_ref, bb3_ref, bw3_ref, _K)

        e1 = jnp.exp(r1)
        p1 = e1 / jnp.sum(e1, axis=-1, keepdims=True)

        g16, b16 = _group_mats(_ND * _NS, _NS)
        e2 = jnp.exp(r2)
        s2 = jnp.dot(e2.astype(jnp.bfloat16), g16, preferred_element_type=jnp.float32)
        p2 = e2 / s2
        j2 = jnp.dot(p1.astype(jnp.bfloat16), b16, preferred_element_type=jnp.float32) * p2

        g64, b64 = _group_mats(_K, _NC)
        e3 = jnp.exp(r3)
        s3 = jnp.dot(e3.astype(jnp.bfloat16), g64, preferred_element_type=jnp.float32)
        p3 = e3 / s3
        j3 = jnp.dot(j2.astype(jnp.bfloat16), b64, preferred_element_type=jnp.float32) * p3
        j3_scr[...] = j3.astype(jnp.bfloat16)

    we = we_ref[0].astype(jnp.float8_e4m3fn)
    mm = jnp.dot(xb_scr[...], we, preferred_element_type=jnp.float32)
    oh = (jax.lax.broadcasted_iota(jnp.int32, (_K, 1), 0) == k
          ).astype(jnp.bfloat16)
    col = jnp.dot(j3_scr[...], oh, preferred_element_type=jnp.float32)
    contrib = col * mm

    @pl.when(k == 0)
    def _first():
        out_ref[...] = contrib

    @pl.when(k > 0)
    def _accum():
        out_ref[...] = out_ref[...] + contrib

    @pl.when(k == _K - 1)
    def _finish():
        h = out_ref[...] + jnp.dot(
            j3_scr[...], be_ref[...].astype(jnp.bfloat16),
            preferred_element_type=jnp.float32)
        y = x_ref[...] + jax.nn.gelu(h)
        mu = jnp.mean(y, axis=-1, keepdims=True)
        var = jnp.mean((y - mu) ** 2, axis=-1, keepdims=True)
        out_ref[...] = ((y - mu) * jax.lax.rsqrt(var + 1e-5)
                        * gamma_ref[...] + beta_ref[...])


def kernel(x, W_color, b_color, Wb1, bb1, bw1, Wb2, bb2, bw2, Wb3, bb3, bw3,
           We, be, gamma, beta):
    x2 = x.reshape(_S, _D)
    out = pl.pallas_call(
        _body,
        grid=(_K,),
        in_specs=[
            pl.BlockSpec((_S, _D), lambda k: (0, 0)),
            pl.BlockSpec((_D, _SD), lambda k: (0, 0)),
            pl.BlockSpec((1, _SD), lambda k: (0, 0)),
            pl.BlockSpec((_NB, _BSZ, _ND), lambda k: (0, 0, 0)),
            pl.BlockSpec((_NB, _ND), lambda k: (0, 0)),
            pl.BlockSpec((1, _NB), lambda k: (0, 0)),
            pl.BlockSpec((_NB, _BSZ, _ND * _NS), lambda k: (0, 0, 0)),
            pl.BlockSpec((_NB, _ND * _NS), lambda k: (0, 0)),
            pl.BlockSpec((1, _NB), lambda k: (0, 0)),
            pl.BlockSpec((_NB, _BSZ, _K), lambda k: (0, 0, 0)),
            pl.BlockSpec((_NB, _K), lambda k: (0, 0)),
            pl.BlockSpec((1, _NB), lambda k: (0, 0)),
            pl.BlockSpec((1, _D, _D), lambda k: (k, 0, 0)),
            pl.BlockSpec((_K, _D), lambda k: (0, 0)),
            pl.BlockSpec((1, _D), lambda k: (0, 0)),
            pl.BlockSpec((1, _D), lambda k: (0, 0)),
        ],
        out_specs=pl.BlockSpec((_S, _D), lambda k: (0, 0)),
        out_shape=jax.ShapeDtypeStruct((_S, _D), jnp.float32),
        scratch_shapes=[
            pltpu.VMEM((_S, _D), jnp.float8_e4m3fn),
            pltpu.VMEM((_S, _K), jnp.bfloat16),
        ],
        compiler_params=pltpu.CompilerParams(
            dimension_semantics=("arbitrary",),
        ),
    )(x2, W_color, b_color.reshape(1, _SD), Wb1, bb1, bw1.reshape(1, _NB),
      Wb2, bb2, bw2.reshape(1, _NB), Wb3, bb3, bw3.reshape(1, _NB),
      We, be, gamma.reshape(1, _D), beta.reshape(1, _D))
    return out.reshape(1, _S, _D)


# speedup vs baseline: 3.5848x; 1.0735x over previous
import jax
import jax.numpy as jnp
from jax.experimental import pallas as pl
from jax.experimental.pallas import tpu as pltpu

_S, _D, _SD = 2048, 1024, 64
_ND, _NS, _NC = 4, 4, 4
_NB, _BSZ = 4, 16
_K = _ND * _NS * _NC
_EG = 2


def _group_mats(width, group):
    i = jax.lax.broadcasted_iota(jnp.int32, (width, width), 0)
    j = jax.lax.broadcasted_iota(jnp.int32, (width, width), 1)
    g = jnp.where((i // group) == (j // group), 1.0, 0.0).astype(jnp.bfloat16)
    gi = jax.lax.broadcasted_iota(jnp.int32, (width // group, width), 0)
    gj = jax.lax.broadcasted_iota(jnp.int32, (width // group, width), 1)
    b = jnp.where(gi == (gj // group), 1.0, 0.0).astype(jnp.bfloat16)
    return g, b


def _body(x_ref, wcol_ref, bcol_ref, wb1_ref, bb1_ref, bw1_ref,
          wb2_ref, bb2_ref, bw2_ref, wb3_ref, bb3_ref, bw3_ref,
          we_ref, be_ref, gamma_ref, beta_ref,
          out_ref, xb_scr, j3_scr):
    k = pl.program_id(0)

    @pl.when(k == 0)
    def _routing():
        xb = x_ref[...].astype(jnp.bfloat16)
        xb_scr[...] = x_ref[...].astype(jnp.float8_e4m3fn)
        wcol = wcol_ref[...].astype(jnp.bfloat16)
        color = jnp.tanh(
            jnp.dot(xb, wcol, preferred_element_type=jnp.float32)
            + bcol_ref[...])
        color_b = color.astype(jnp.bfloat16)

        def chroma(wb_ref, bb_ref, bw_ref, kl):
            bw = bw_ref[...]
            e = jnp.exp(bw - jnp.max(bw, axis=-1, keepdims=True))
            w = e / jnp.sum(e, axis=-1, keepdims=True)
            r = jnp.zeros((_S, kl), jnp.float32)
            for n in range(_NB):
                band = color_b[:, n * _BSZ:(n + 1) * _BSZ]
                wn = wb_ref[n].astype(jnp.bfloat16)
                logit = jnp.dot(band, wn, preferred_element_type=jnp.float32)
                logit = logit + bb_ref[n:n + 1, :]
                r = r + w[:, n:n + 1] * jax.nn.sigmoid(logit)
            return r

        r1 = chroma(wb1_ref, bb1_ref, bw1_ref, _ND)
        r2 = chroma(wb2_ref, bb2_ref, bw2_ref, _ND * _NS)
        r3 = chroma(wb3_ref, bb3_ref, bw3_ref, _K)

        e1 = jnp.exp(r1)
        p1 = e1 / jnp.sum(e1, axis=-1, keepdims=True)

        g16, b16 = _group_mats(_ND * _NS, _NS)
        e2 = jnp.exp(r2)
        s2 = jnp.dot(e2.astype(jnp.bfloat16), g16, preferred_element_type=jnp.float32)
        p2 = e2 / s2
        j2 = jnp.dot(p1.astype(jnp.bfloat16), b16, preferred_element_type=jnp.float32) * p2

        g64, b64 = _group_mats(_K, _NC)
        e3 = jnp.exp(r3)
        s3 = jnp.dot(e3.astype(jnp.bfloat16), g64, preferred_element_type=jnp.float32)
        p3 = e3 / s3
        j3 = jnp.dot(j2.astype(jnp.bfloat16), b64, preferred_element_type=jnp.float32) * p3
        j3_scr[...] = j3.astype(jnp.bfloat16)

    ki = jax.lax.broadcasted_iota(jnp.int32, (_K, _EG), 0)
    kj = jax.lax.broadcasted_iota(jnp.int32, (_K, _EG), 1)
    oh = (ki == k * _EG + kj).astype(jnp.bfloat16)
    cols = jnp.dot(j3_scr[...], oh,
                   preferred_element_type=jnp.float32)
    contrib = None
    for i in range(_EG):
        we = we_ref[i].astype(jnp.float8_e4m3fn)
        mm = jnp.dot(xb_scr[...], we, preferred_element_type=jnp.float32)
        term = cols[:, i:i + 1] * mm
        contrib = term if contrib is None else contrib + term

    @pl.when(k == 0)
    def _first():
        out_ref[...] = contrib

    @pl.when(k > 0)
    def _accum():
        out_ref[...] = out_ref[...] + contrib

    @pl.when(k == _K // _EG - 1)
    def _finish():
        h = out_ref[...] + jnp.dot(
            j3_scr[...], be_ref[...].astype(jnp.bfloat16),
            preferred_element_type=jnp.float32)
        y = x_ref[...] + jax.nn.gelu(h)
        mu = jnp.mean(y, axis=-1, keepdims=True)
        var = jnp.mean((y - mu) ** 2, axis=-1, keepdims=True)
        out_ref[...] = ((y - mu) * jax.lax.rsqrt(var + 1e-5)
                        * gamma_ref[...] + beta_ref[...])


def kernel(x, W_color, b_color, Wb1, bb1, bw1, Wb2, bb2, bw2, Wb3, bb3, bw3,
           We, be, gamma, beta):
    x2 = x.reshape(_S, _D)
    out = pl.pallas_call(
        _body,
        grid=(_K // _EG,),
        in_specs=[
            pl.BlockSpec((_S, _D), lambda k: (0, 0)),
            pl.BlockSpec((_D, _SD), lambda k: (0, 0)),
            pl.BlockSpec((1, _SD), lambda k: (0, 0)),
            pl.BlockSpec((_NB, _BSZ, _ND), lambda k: (0, 0, 0)),
            pl.BlockSpec((_NB, _ND), lambda k: (0, 0)),
            pl.BlockSpec((1, _NB), lambda k: (0, 0)),
            pl.BlockSpec((_NB, _BSZ, _ND * _NS), lambda k: (0, 0, 0)),
            pl.BlockSpec((_NB, _ND * _NS), lambda k: (0, 0)),
            pl.BlockSpec((1, _NB), lambda k: (0, 0)),
            pl.BlockSpec((_NB, _BSZ, _K), lambda k: (0, 0, 0)),
            pl.BlockSpec((_NB, _K), lambda k: (0, 0)),
            pl.BlockSpec((1, _NB), lambda k: (0, 0)),
            pl.BlockSpec((_EG, _D, _D), lambda k: (k, 0, 0)),
            pl.BlockSpec((_K, _D), lambda k: (0, 0)),
            pl.BlockSpec((1, _D), lambda k: (0, 0)),
            pl.BlockSpec((1, _D), lambda k: (0, 0)),
        ],
        out_specs=pl.BlockSpec((_S, _D), lambda k: (0, 0)),
        out_shape=jax.ShapeDtypeStruct((_S, _D), jnp.float32),
        scratch_shapes=[
            pltpu.VMEM((_S, _D), jnp.float8_e4m3fn),
            pltpu.VMEM((_S, _K), jnp.bfloat16),
        ],
        compiler_params=pltpu.CompilerParams(
            dimension_semantics=("arbitrary",),
        ),
    )(x2, W_color, b_color.reshape(1, _SD), Wb1, bb1, bw1.reshape(1, _NB),
      Wb2, bb2, bw2.reshape(1, _NB), Wb3, bb3, bw3.reshape(1, _NB),
      We, be, gamma.reshape(1, _D), beta.reshape(1, _D))
    return out.reshape(1, _S, _D)


# speedup vs baseline: 3.7810x; 1.0548x over previous
import jax
import jax.numpy as jnp
from jax.experimental import pallas as pl
from jax.experimental.pallas import tpu as pltpu

_S, _D, _SD = 2048, 1024, 64
_ND, _NS, _NC = 4, 4, 4
_NB, _BSZ = 4, 16
_K = _ND * _NS * _NC
_EG = 2


def _group_mats(width, group):
    i = jax.lax.broadcasted_iota(jnp.int32, (width, width), 0)
    j = jax.lax.broadcasted_iota(jnp.int32, (width, width), 1)
    g = jnp.where((i // group) == (j // group), 1.0, 0.0).astype(jnp.bfloat16)
    gi = jax.lax.broadcasted_iota(jnp.int32, (width // group, width), 0)
    gj = jax.lax.broadcasted_iota(jnp.int32, (width // group, width), 1)
    b = jnp.where(gi == (gj // group), 1.0, 0.0).astype(jnp.bfloat16)
    return g, b


def _body(x_ref, wcol_ref, bcol_ref, wb1_ref, bb1_ref, bw1_ref,
          wb2_ref, bb2_ref, bw2_ref, wb3_ref, bb3_ref, bw3_ref,
          we_ref, be_ref, gamma_ref, beta_ref,
          out_ref, xb_scr, j3_scr):
    k = pl.program_id(0)

    @pl.when(k == 0)
    def _routing():
        xb = x_ref[...].astype(jnp.bfloat16)
        xb_scr[...] = x_ref[...].astype(jnp.float8_e4m3fn)
        wcol = wcol_ref[...].astype(jnp.bfloat16)
        color = jnp.tanh(
            jnp.dot(xb, wcol, preferred_element_type=jnp.float32)
            + bcol_ref[...])
        color_b = color.astype(jnp.bfloat16)

        def chroma(wb_ref, bb_ref, bw_ref, kl):
            bw = bw_ref[...]
            e = jnp.exp(bw - jnp.max(bw, axis=-1, keepdims=True))
            w = e / jnp.sum(e, axis=-1, keepdims=True)
            r = jnp.zeros((_S, kl), jnp.float32)
            for n in range(_NB):
                band = color_b[:, n * _BSZ:(n + 1) * _BSZ]
                wn = wb_ref[n].astype(jnp.bfloat16)
                logit = jnp.dot(band, wn, preferred_element_type=jnp.float32)
                logit = logit + bb_ref[n:n + 1, :]
                r = r + w[:, n:n + 1] * jax.nn.sigmoid(logit)
            return r

        r1 = chroma(wb1_ref, bb1_ref, bw1_ref, _ND)
        r2 = chroma(wb2_ref, bb2_ref, bw2_ref, _ND * _NS)
        r3 = chroma(wb3_ref, bb3_ref, bw3_ref, _K)

        e1 = jnp.exp(r1)
        p1 = e1 / jnp.sum(e1, axis=-1, keepdims=True)

        g16, b16 = _group_mats(_ND * _NS, _NS)
        e2 = jnp.exp(r2)
        s2 = jnp.dot(e2.astype(jnp.bfloat16), g16, preferred_element_type=jnp.float32)
        p2 = e2 / s2
        j2 = jnp.dot(p1.astype(jnp.bfloat16), b16, preferred_element_type=jnp.float32) * p2

        g64, b64 = _group_mats(_K, _NC)
        e3 = jnp.exp(r3)
        s3 = jnp.dot(e3.astype(jnp.bfloat16), g64, preferred_element_type=jnp.float32)
        p3 = e3 / s3
        j3 = jnp.dot(j2.astype(jnp.bfloat16), b64, preferred_element_type=jnp.float32) * p3
        j3_scr[...] = j3.astype(jnp.bfloat16)

    ki = jax.lax.broadcasted_iota(jnp.int32, (_K, _EG), 0)
    kj = jax.lax.broadcasted_iota(jnp.int32, (_K, _EG), 1)
    oh = (ki == k * _EG + kj).astype(jnp.bfloat16)
    cols = jnp.dot(j3_scr[...], oh,
                   preferred_element_type=jnp.float32)
    mms = []
    for i in range(_EG):
        we = we_ref[i].astype(jnp.float8_e4m3fn)
        mms.append(jnp.dot(xb_scr[...], we, preferred_element_type=jnp.float32))

    @pl.when(k == 0)
    def _first():
        out_ref[...] = cols[:, 0:1] * mms[0] + cols[:, 1:2] * mms[1]

    @pl.when(k > 0)
    def _accum():
        out_ref[...] = (out_ref[...] + cols[:, 0:1] * mms[0]
                        + cols[:, 1:2] * mms[1])

    @pl.when(k == _K // _EG - 1)
    def _finish():
        h = out_ref[...] + jnp.dot(
            j3_scr[...], be_ref[...].astype(jnp.bfloat16),
            preferred_element_type=jnp.float32)
        y = x_ref[...] + jax.nn.gelu(h)
        mu = jnp.mean(y, axis=-1, keepdims=True)
        var = jnp.mean((y - mu) ** 2, axis=-1, keepdims=True)
        out_ref[...] = ((y - mu) * jax.lax.rsqrt(var + 1e-5)
                        * gamma_ref[...] + beta_ref[...])


def kernel(x, W_color, b_color, Wb1, bb1, bw1, Wb2, bb2, bw2, Wb3, bb3, bw3,
           We, be, gamma, beta):
    x2 = x.reshape(_S, _D)
    out = pl.pallas_call(
        _body,
        grid=(_K // _EG,),
        in_specs=[
            pl.BlockSpec((_S, _D), lambda k: (0, 0)),
            pl.BlockSpec((_D, _SD), lambda k: (0, 0)),
            pl.BlockSpec((1, _SD), lambda k: (0, 0)),
            pl.BlockSpec((_NB, _BSZ, _ND), lambda k: (0, 0, 0)),
            pl.BlockSpec((_NB, _ND), lambda k: (0, 0)),
            pl.BlockSpec((1, _NB), lambda k: (0, 0)),
            pl.BlockSpec((_NB, _BSZ, _ND * _NS), lambda k: (0, 0, 0)),
            pl.BlockSpec((_NB, _ND * _NS), lambda k: (0, 0)),
            pl.BlockSpec((1, _NB), lambda k: (0, 0)),
            pl.BlockSpec((_NB, _BSZ, _K), lambda k: (0, 0, 0)),
            pl.BlockSpec((_NB, _K), lambda k: (0, 0)),
            pl.BlockSpec((1, _NB), lambda k: (0, 0)),
            pl.BlockSpec((_EG, _D, _D), lambda k: (k, 0, 0)),
            pl.BlockSpec((_K, _D), lambda k: (0, 0)),
            pl.BlockSpec((1, _D), lambda k: (0, 0)),
            pl.BlockSpec((1, _D), lambda k: (0, 0)),
        ],
        out_specs=pl.BlockSpec((_S, _D), lambda k: (0, 0)),
        out_shape=jax.ShapeDtypeStruct((_S, _D), jnp.float32),
        scratch_shapes=[
            pltpu.VMEM((_S, _D), jnp.float8_e4m3fn),
            pltpu.VMEM((_S, _K), jnp.bfloat16),
        ],
        compiler_params=pltpu.CompilerParams(
            dimension_semantics=("arbitrary",),
            vmem_limit_bytes=63 * 1024 * 1024,
        ),
    )(x2, W_color, b_color.reshape(1, _SD), Wb1, bb1, bw1.reshape(1, _NB),
      Wb2, bb2, bw2.reshape(1, _NB), Wb3, bb3, bw3.reshape(1, _NB),
      We, be, gamma.reshape(1, _D), beta.reshape(1, _D))
    return out.reshape(1, _S, _D)
